# baseline (speedup 1.0000x reference)
import functools

import jax
import jax.numpy as jnp
from jax import lax
from jax.experimental import pallas as pl
from jax.experimental.pallas import tpu as pltpu
from jax.experimental.pallas import tpu_sc as plsc

H = 384
W = 384
C = 96
A = 4096
NB = 2
K = 9
ROWS = A * K
CP = 128

_NW = 32
_CHUNK = 128
_ROWS_PER_W = ROWS // _NW
_NCHUNK = _ROWS_PER_W // _CHUNK


def _sc_gather_body(t_hbm, idx_hbm, out_hbm, idx_v, buf0, buf1, sem0, sem1):
    wid = lax.axis_index("s") * 2 + lax.axis_index("c")
    pltpu.sync_copy(idx_hbm.at[wid], idx_v)
    bufs = (buf0, buf1)
    sems = (sem0, sem1)
    cps = [None, None]
    for j in range(_NCHUNK):
        p = j % 2
        if cps[p] is not None:
            cps[p].wait()
            base = wid * _ROWS_PER_W + (j - 2) * _CHUNK
            pltpu.sync_copy(bufs[p], out_hbm.at[pl.ds(base, _CHUNK)])
        cps[p] = pltpu.async_copy(t_hbm.at[idx_v.at[j]], bufs[p], sems[p])
    for j in (_NCHUNK - 2, _NCHUNK - 1):
        p = j % 2
        cps[p].wait()
        base = wid * _ROWS_PER_W + j * _CHUNK
        pltpu.sync_copy(bufs[p], out_hbm.at[pl.ds(base, _CHUNK)])


@functools.cache
def _get_sc_gather():
    return pl.kernel(
        _sc_gather_body,
        out_type=jax.ShapeDtypeStruct((ROWS, CP), jnp.uint32),
        mesh=plsc.VectorSubcoreMesh(core_axis_name="c", subcore_axis_name="s"),
        scratch_types=[
            pltpu.VMEM((_NCHUNK, _CHUNK), jnp.int32),
            pltpu.VMEM((_CHUNK, CP), jnp.uint32),
            pltpu.VMEM((_CHUNK, CP), jnp.uint32),
            pltpu.SemaphoreType.DMA,
            pltpu.SemaphoreType.DMA,
        ],
    )


_BH = 16
_NH = H // _BH


def _tc_transpose_body(f1_ref, f2_ref, out_ref):
    eye = jnp.eye(C, dtype=jnp.bfloat16)

    def slab_bits(f_ref):
        b = f_ref[...].reshape(C, _BH * W).astype(jnp.bfloat16)
        xt = lax.dot_general(b, eye, (((0,), (0,)), ((), ())),
                             preferred_element_type=jnp.float32)
        return lax.bitcast_convert_type(xt, jnp.uint32) >> 16

    w = slab_bits(f1_ref) | (slab_bits(f2_ref) << 16)
    out_ref[...] = jnp.concatenate(
        [w, jnp.zeros((_BH * W, CP - C), jnp.uint32)], axis=1)


def _tc_transpose(f1, f2):
    return pl.pallas_call(
        _tc_transpose_body,
        grid=(_NH,),
        in_specs=[
            pl.BlockSpec((C, _BH, W), lambda h: (0, h, 0)),
            pl.BlockSpec((C, _BH, W), lambda h: (0, h, 0)),
        ],
        out_specs=pl.BlockSpec((_BH * W, CP), lambda h: (h, 0)),
        out_shape=jax.ShapeDtypeStruct((H * W, CP), jnp.uint32),
    )(f1, f2)


_ABLK = 512
_NBLK = A // _ABLK


def _unpack(w):
    g1 = lax.bitcast_convert_type(w << 16, jnp.float32)
    g2 = lax.bitcast_convert_type(w & jnp.uint32(0xFFFF0000), jnp.float32)
    return g1, g2


def _tc_cosine_body(c_ref, n_ref, out_ref):
    ones = jnp.ones((CP, 128), jnp.bfloat16)

    def rowsums(p):
        return lax.dot_general(p.astype(jnp.bfloat16), ones,
                               (((1,), (0,)), ((), ())),
                               preferred_element_type=jnp.float32)

    def unit(g):
        return g * lax.rsqrt(rowsums(g * g))

    c1, c2 = _unpack(c_ref[...])
    n1, n2 = _unpack(n_ref[...])

    def sims(c, n):
        cb = jnp.broadcast_to(unit(c)[:, None, :], (_ABLK, K - 1, CP))
        pd = unit(n) * cb.reshape(_ABLK * (K - 1), CP)
        return rowsums(pd)

    part = jnp.sum(jnp.abs(sims(c1, n1) - sims(c2, n2))) / 128.0
    out_ref[pl.ds(pl.program_id(0), 1), :] = jnp.full((1, 128), part, jnp.float32)


def _tc_cosine(g):
    out = pl.pallas_call(
        _tc_cosine_body,
        grid=(_NBLK,),
        in_specs=[
            pl.BlockSpec((_ABLK, CP), lambda i: (i, 0)),
            pl.BlockSpec((_ABLK * (K - 1), CP), lambda i: (i + 1, 0)),
        ],
        out_specs=pl.BlockSpec((_NBLK, 128), lambda i: (0, 0)),
        out_shape=jax.ShapeDtypeStruct((_NBLK, 128), jnp.float32),
    )(g, g)
    return jnp.sum(out[:, 0])


def kernel(feat_list_1, feat_list_2, index_list):
    n = feat_list_1.shape[0]
    total = jnp.float32(0.0)
    for i in range(n):
        idx = index_list[i].astype(jnp.int32)
        q = idx[..., 0] * W + idx[..., 1]
        for b in range(NB):
            qb = jnp.concatenate(
                [q[b, :, 0].reshape(-1), q[b, :, 1:].reshape(-1)])
            qb = qb.reshape(_NW, _NCHUNK, _CHUNK)
            t = _tc_transpose(feat_list_1[i, b], feat_list_2[i, b])
            g = _get_sc_gather()(t, qb)
            total = total + _tc_cosine(g) / (NB * A * 8)
    return total / n


# speedup vs baseline: 1.5911x; 1.5911x over previous
import functools

import jax
import jax.numpy as jnp
from jax import lax
from jax.experimental import pallas as pl
from jax.experimental.pallas import tpu as pltpu
from jax.experimental.pallas import tpu_sc as plsc

H = 384
W = 384
C = 96
A = 4096
NB = 2
K = 9
ROWS = NB * A * K
CP = 128

_NW = 32
_CHUNK = 128
_ROWS_PER_W = ROWS // _NW
_NCHUNK = _ROWS_PER_W // _CHUNK


def _sc_gather_body(t_hbm, idx_hbm, out_hbm, idx_v, buf0, buf1, sem0, sem1):
    wid = lax.axis_index("s") * 2 + lax.axis_index("c")
    pltpu.sync_copy(idx_hbm.at[wid], idx_v)
    bufs = (buf0, buf1)
    sems = (sem0, sem1)
    cps = [None, None]
    for j in range(_NCHUNK):
        p = j % 2
        if cps[p] is not None:
            cps[p].wait()
            base = wid * _ROWS_PER_W + (j - 2) * _CHUNK
            pltpu.sync_copy(bufs[p], out_hbm.at[pl.ds(base, _CHUNK)])
        cps[p] = pltpu.async_copy(t_hbm.at[idx_v.at[j]], bufs[p], sems[p])
    for j in (_NCHUNK - 2, _NCHUNK - 1):
        p = j % 2
        cps[p].wait()
        base = wid * _ROWS_PER_W + j * _CHUNK
        pltpu.sync_copy(bufs[p], out_hbm.at[pl.ds(base, _CHUNK)])


@functools.cache
def _get_sc_gather():
    return pl.kernel(
        _sc_gather_body,
        out_type=jax.ShapeDtypeStruct((ROWS, CP), jnp.uint32),
        mesh=plsc.VectorSubcoreMesh(core_axis_name="c", subcore_axis_name="s"),
        scratch_types=[
            pltpu.VMEM((_NCHUNK, _CHUNK), jnp.int32),
            pltpu.VMEM((_CHUNK, CP), jnp.uint32),
            pltpu.VMEM((_CHUNK, CP), jnp.uint32),
            pltpu.SemaphoreType.DMA,
            pltpu.SemaphoreType.DMA,
        ],
    )


_BH = 16
_NH = H // _BH


def _tc_transpose_body(f1_ref, f2_ref, out_ref):
    eye = jnp.eye(C, dtype=jnp.bfloat16)

    def slab_bits(f_ref):
        b = f_ref[0].reshape(C, _BH * W).astype(jnp.bfloat16)
        xt = lax.dot_general(b, eye, (((0,), (0,)), ((), ())),
                             preferred_element_type=jnp.float32)
        return lax.bitcast_convert_type(xt, jnp.uint32) >> 16

    w = slab_bits(f1_ref) | (slab_bits(f2_ref) << 16)
    out_ref[...] = jnp.concatenate(
        [w, jnp.zeros((_BH * W, CP - C), jnp.uint32)], axis=1)


def _tc_transpose(f1, f2):
    return pl.pallas_call(
        _tc_transpose_body,
        grid=(NB, _NH),
        in_specs=[
            pl.BlockSpec((1, C, _BH, W), lambda b, h: (b, 0, h, 0)),
            pl.BlockSpec((1, C, _BH, W), lambda b, h: (b, 0, h, 0)),
        ],
        out_specs=pl.BlockSpec((_BH * W, CP), lambda b, h: (b * _NH + h, 0)),
        out_shape=jax.ShapeDtypeStruct((NB * H * W, CP), jnp.uint32),
    )(f1, f2)


_ABLK = 512
_NBLK = NB * A // _ABLK


def _unpack(w):
    g1 = lax.bitcast_convert_type(w << 16, jnp.float32)
    g2 = lax.bitcast_convert_type(w & jnp.uint32(0xFFFF0000), jnp.float32)
    return g1, g2


def _tc_cosine_body(c_ref, n_ref, out_ref):
    ones = jnp.ones((CP, 128), jnp.bfloat16)

    def rowsums(p):
        return lax.dot_general(p.astype(jnp.bfloat16), ones,
                               (((1,), (0,)), ((), ())),
                               preferred_element_type=jnp.float32)

    def unit(g):
        return g * lax.rsqrt(rowsums(g * g))

    c1, c2 = _unpack(c_ref[...])
    n1, n2 = _unpack(n_ref[...])

    def sims(c, n):
        cb = jnp.broadcast_to(unit(c)[:, None, :], (_ABLK, K - 1, CP))
        pd = unit(n) * cb.reshape(_ABLK * (K - 1), CP)
        return rowsums(pd)

    part = jnp.sum(jnp.abs(sims(c1, n1) - sims(c2, n2))) / 128.0
    out_ref[pl.ds(pl.program_id(0), 1), :] = jnp.full((1, 128), part, jnp.float32)


def _tc_cosine(g):
    out = pl.pallas_call(
        _tc_cosine_body,
        grid=(_NBLK,),
        in_specs=[
            pl.BlockSpec((_ABLK, CP), lambda i: (i, 0)),
            pl.BlockSpec((_ABLK * (K - 1), CP), lambda i: (i + _NBLK // 8, 0)),
        ],
        out_specs=pl.BlockSpec((_NBLK, 128), lambda i: (0, 0)),
        out_shape=jax.ShapeDtypeStruct((_NBLK, 128), jnp.float32),
    )(g, g)
    return jnp.sum(out[:, 0])


def kernel(feat_list_1, feat_list_2, index_list):
    n = feat_list_1.shape[0]
    total = jnp.float32(0.0)
    for i in range(n):
        idx = index_list[i].astype(jnp.int32)
        q = idx[..., 0] * W + idx[..., 1]
        q = q + (jnp.arange(NB, dtype=jnp.int32) * (H * W)).reshape(NB, 1, 1)
        q = jnp.concatenate([q[..., 0].reshape(-1), q[..., 1:].reshape(-1)])
        q = q.reshape(_NW, _NCHUNK, _CHUNK)
        t = _tc_transpose(feat_list_1[i], feat_list_2[i])
        g = _get_sc_gather()(t, q)
        total = total + _tc_cosine(g) / (NB * A * 8)
    return total / n


# speedup vs baseline: 1.6240x; 1.0207x over previous
import functools

import jax
import jax.numpy as jnp
from jax import lax
from jax.experimental import pallas as pl
from jax.experimental.pallas import tpu as pltpu
from jax.experimental.pallas import tpu_sc as plsc

H = 384
W = 384
C = 96
A = 4096
NB = 2
K = 9
ROWS = NB * A * K
CP = 128

_NW = 32
_CHUNK = 128
_ROWS_PER_W = ROWS // _NW
_NCHUNK = _ROWS_PER_W // _CHUNK


def _sc_gather_body(t_hbm, idx_hbm, out_hbm, idx_v, buf0, buf1, sem0, sem1):
    wid = lax.axis_index("s") * 2 + lax.axis_index("c")
    pltpu.sync_copy(idx_hbm.at[wid], idx_v)
    bufs = (buf0, buf1)
    sems = (sem0, sem1)
    cps = [None, None]
    for j in range(_NCHUNK):
        p = j % 2
        if cps[p] is not None:
            cps[p].wait()
            base = wid * _ROWS_PER_W + (j - 2) * _CHUNK
            pltpu.sync_copy(bufs[p], out_hbm.at[pl.ds(base, _CHUNK)])
        cps[p] = pltpu.async_copy(t_hbm.at[idx_v.at[j]], bufs[p], sems[p])
    for j in (_NCHUNK - 2, _NCHUNK - 1):
        p = j % 2
        cps[p].wait()
        base = wid * _ROWS_PER_W + j * _CHUNK
        pltpu.sync_copy(bufs[p], out_hbm.at[pl.ds(base, _CHUNK)])


@functools.cache
def _get_sc_gather():
    return pl.kernel(
        _sc_gather_body,
        out_type=jax.ShapeDtypeStruct((ROWS, CP), jnp.uint32),
        mesh=plsc.VectorSubcoreMesh(core_axis_name="c", subcore_axis_name="s"),
        scratch_types=[
            pltpu.VMEM((_NCHUNK, _CHUNK), jnp.int32),
            pltpu.VMEM((_CHUNK, CP), jnp.uint32),
            pltpu.VMEM((_CHUNK, CP), jnp.uint32),
            pltpu.SemaphoreType.DMA,
            pltpu.SemaphoreType.DMA,
        ],
    )


_BH = 32
_NH = H // _BH


def _tc_transpose_body(f1_ref, f2_ref, out_ref):
    eye = jnp.eye(C, dtype=jnp.bfloat16)

    def slab_bits(f_ref):
        b = f_ref[0].reshape(C, _BH * W).astype(jnp.bfloat16)
        xt = lax.dot_general(b, eye, (((0,), (0,)), ((), ())),
                             preferred_element_type=jnp.float32)
        return lax.bitcast_convert_type(xt, jnp.uint32) >> 16

    w = slab_bits(f1_ref) | (slab_bits(f2_ref) << 16)
    out_ref[...] = jnp.concatenate(
        [w, jnp.zeros((_BH * W, CP - C), jnp.uint32)], axis=1)


def _tc_transpose(f1, f2):
    return pl.pallas_call(
        _tc_transpose_body,
        grid=(NB, _NH),
        in_specs=[
            pl.BlockSpec((1, C, _BH, W), lambda b, h: (b, 0, h, 0)),
            pl.BlockSpec((1, C, _BH, W), lambda b, h: (b, 0, h, 0)),
        ],
        out_specs=pl.BlockSpec((_BH * W, CP), lambda b, h: (b * _NH + h, 0)),
        out_shape=jax.ShapeDtypeStruct((NB * H * W, CP), jnp.uint32),
    )(f1, f2)


_ABLK = 512
_NBLK = NB * A // _ABLK


def _unpack(w):
    g1 = lax.bitcast_convert_type(w << 16, jnp.float32)
    g2 = lax.bitcast_convert_type(w & jnp.uint32(0xFFFF0000), jnp.float32)
    return g1, g2


def _tc_cosine_body(c_ref, n_ref, out_ref):
    ones = jnp.ones((CP, 128), jnp.bfloat16)

    def rowsums(p):
        return lax.dot_general(p.astype(jnp.bfloat16), ones,
                               (((1,), (0,)), ((), ())),
                               preferred_element_type=jnp.float32)

    def unit(g):
        return g * lax.rsqrt(rowsums(g * g))

    c1, c2 = _unpack(c_ref[...])
    n1, n2 = _unpack(n_ref[...])

    def sims(c, n):
        cb = jnp.broadcast_to(unit(c)[:, None, :], (_ABLK, K - 1, CP))
        pd = unit(n) * cb.reshape(_ABLK * (K - 1), CP)
        return rowsums(pd)

    part = jnp.sum(jnp.abs(sims(c1, n1) - sims(c2, n2))) / 128.0
    out_ref[pl.ds(pl.program_id(0), 1), :] = jnp.full((1, 128), part, jnp.float32)


def _tc_cosine(g):
    out = pl.pallas_call(
        _tc_cosine_body,
        grid=(_NBLK,),
        in_specs=[
            pl.BlockSpec((_ABLK, CP), lambda i: (i, 0)),
            pl.BlockSpec((_ABLK * (K - 1), CP), lambda i: (i + _NBLK // 8, 0)),
        ],
        out_specs=pl.BlockSpec((_NBLK, 128), lambda i: (0, 0)),
        out_shape=jax.ShapeDtypeStruct((_NBLK, 128), jnp.float32),
    )(g, g)
    return jnp.sum(out[:, 0])


def kernel(feat_list_1, feat_list_2, index_list):
    n = feat_list_1.shape[0]
    total = jnp.float32(0.0)
    for i in range(n):
        idx = index_list[i].astype(jnp.int32)
        q = idx[..., 0] * W + idx[..., 1]
        q = q + (jnp.arange(NB, dtype=jnp.int32) * (H * W)).reshape(NB, 1, 1)
        q = jnp.concatenate([q[..., 0].reshape(-1), q[..., 1:].reshape(-1)])
        q = q.reshape(_NW, _NCHUNK, _CHUNK)
        t = _tc_transpose(feat_list_1[i], feat_list_2[i])
        g = _get_sc_gather()(t, q)
        total = total + _tc_cosine(g) / (NB * A * 8)
    return total / n


# speedup vs baseline: 1.6491x; 1.0155x over previous
import functools

import jax
import jax.numpy as jnp
from jax import lax
from jax.experimental import pallas as pl
from jax.experimental.pallas import tpu as pltpu
from jax.experimental.pallas import tpu_sc as plsc

H = 384
W = 384
C = 96
A = 4096
NB = 2
K = 9
ROWS = NB * A * K
CP = 128

_NW = 32
_CHUNK = 128
_ROWS_PER_W = ROWS // _NW
_NCHUNK = _ROWS_PER_W // _CHUNK


def _sc_gather_body(t_hbm, idx_hbm, out_hbm, idx_v, buf0, buf1, sem0, sem1):
    wid = lax.axis_index("s") * 2 + lax.axis_index("c")
    pltpu.sync_copy(idx_hbm.at[wid], idx_v)
    bufs = (buf0, buf1)
    sems = (sem0, sem1)
    cps = [None, None]
    for j in range(_NCHUNK):
        p = j % 2
        if cps[p] is not None:
            cps[p].wait()
            base = wid * _ROWS_PER_W + (j - 2) * _CHUNK
            pltpu.sync_copy(bufs[p], out_hbm.at[pl.ds(base, _CHUNK)])
        cps[p] = pltpu.async_copy(t_hbm.at[idx_v.at[j]], bufs[p], sems[p])
    for j in (_NCHUNK - 2, _NCHUNK - 1):
        p = j % 2
        cps[p].wait()
        base = wid * _ROWS_PER_W + j * _CHUNK
        pltpu.sync_copy(bufs[p], out_hbm.at[pl.ds(base, _CHUNK)])


@functools.cache
def _get_sc_gather():
    return pl.kernel(
        _sc_gather_body,
        out_type=jax.ShapeDtypeStruct((ROWS, CP), jnp.uint32),
        mesh=plsc.VectorSubcoreMesh(core_axis_name="c", subcore_axis_name="s"),
        scratch_types=[
            pltpu.VMEM((_NCHUNK, _CHUNK), jnp.int32),
            pltpu.VMEM((_CHUNK, CP), jnp.uint32),
            pltpu.VMEM((_CHUNK, CP), jnp.uint32),
            pltpu.SemaphoreType.DMA,
            pltpu.SemaphoreType.DMA,
        ],
    )


_BH = 48
_NH = H // _BH


def _tc_transpose_body(f1_ref, f2_ref, out_ref):
    eye = jnp.eye(C, dtype=jnp.bfloat16)

    def slab_bits(f_ref):
        b = f_ref[0].reshape(C, _BH * W).astype(jnp.bfloat16)
        xt = lax.dot_general(b, eye, (((0,), (0,)), ((), ())),
                             preferred_element_type=jnp.float32)
        return lax.bitcast_convert_type(xt, jnp.uint32) >> 16

    w = slab_bits(f1_ref) | (slab_bits(f2_ref) << 16)
    out_ref[...] = jnp.concatenate(
        [w, jnp.zeros((_BH * W, CP - C), jnp.uint32)], axis=1)


def _tc_transpose(f1, f2):
    return pl.pallas_call(
        _tc_transpose_body,
        grid=(NB, _NH),
        in_specs=[
            pl.BlockSpec((1, C, _BH, W), lambda b, h: (b, 0, h, 0)),
            pl.BlockSpec((1, C, _BH, W), lambda b, h: (b, 0, h, 0)),
        ],
        out_specs=pl.BlockSpec((_BH * W, CP), lambda b, h: (b * _NH + h, 0)),
        out_shape=jax.ShapeDtypeStruct((NB * H * W, CP), jnp.uint32),
    )(f1, f2)


_ABLK = 512
_NBLK = NB * A // _ABLK


def _unpack(w):
    g1 = lax.bitcast_convert_type(w << 16, jnp.float32)
    g2 = lax.bitcast_convert_type(w & jnp.uint32(0xFFFF0000), jnp.float32)
    return g1, g2


def _tc_cosine_body(c_ref, n_ref, out_ref):
    ones = jnp.ones((CP, 128), jnp.bfloat16)

    def rowsums(p):
        return lax.dot_general(p.astype(jnp.bfloat16), ones,
                               (((1,), (0,)), ((), ())),
                               preferred_element_type=jnp.float32)

    def unit(g):
        return g * lax.rsqrt(rowsums(g * g))

    c1, c2 = _unpack(c_ref[...])
    n1, n2 = _unpack(n_ref[...])

    def sims(c, n):
        cb = jnp.broadcast_to(unit(c)[:, None, :], (_ABLK, K - 1, CP))
        pd = unit(n) * cb.reshape(_ABLK * (K - 1), CP)
        return rowsums(pd)

    part = jnp.sum(jnp.abs(sims(c1, n1) - sims(c2, n2))) / 128.0
    out_ref[pl.ds(pl.program_id(0), 1), :] = jnp.full((1, 128), part, jnp.float32)


def _tc_cosine(g):
    out = pl.pallas_call(
        _tc_cosine_body,
        grid=(_NBLK,),
        in_specs=[
            pl.BlockSpec((_ABLK, CP), lambda i: (i, 0)),
            pl.BlockSpec((_ABLK * (K - 1), CP), lambda i: (i + _NBLK // 8, 0)),
        ],
        out_specs=pl.BlockSpec((_NBLK, 128), lambda i: (0, 0)),
        out_shape=jax.ShapeDtypeStruct((_NBLK, 128), jnp.float32),
    )(g, g)
    return jnp.sum(out[:, 0])


def kernel(feat_list_1, feat_list_2, index_list):
    n = feat_list_1.shape[0]
    total = jnp.float32(0.0)
    for i in range(n):
        idx = index_list[i].astype(jnp.int32)
        q = idx[..., 0] * W + idx[..., 1]
        q = q + (jnp.arange(NB, dtype=jnp.int32) * (H * W)).reshape(NB, 1, 1)
        q = jnp.concatenate([q[..., 0].reshape(-1), q[..., 1:].reshape(-1)])
        q = q.reshape(_NW, _NCHUNK, _CHUNK)
        t = _tc_transpose(feat_list_1[i], feat_list_2[i])
        g = _get_sc_gather()(t, q)
        total = total + _tc_cosine(g) / (NB * A * 8)
    return total / n


# speedup vs baseline: 1.6534x; 1.0026x over previous
import functools

import jax
import jax.numpy as jnp
from jax import lax
from jax.experimental import pallas as pl
from jax.experimental.pallas import tpu as pltpu
from jax.experimental.pallas import tpu_sc as plsc

H = 384
W = 384
C = 96
A = 4096
NB = 2
K = 9
ROWS = NB * A * K
CP = 128

_NW = 32
_CHUNK = 128
_ROWS_PER_W = ROWS // _NW
_NCHUNK = _ROWS_PER_W // _CHUNK


_NBUF = 3


def _sc_gather_body(t_hbm, idx_hbm, out_hbm, idx_v, buf0, buf1, buf2,
                    sem0, sem1, sem2):
    wid = lax.axis_index("s") * 2 + lax.axis_index("c")
    pltpu.sync_copy(idx_hbm.at[wid], idx_v)
    bufs = (buf0, buf1, buf2)
    sems = (sem0, sem1, sem2)
    cps = [None] * _NBUF
    for j in range(_NCHUNK):
        p = j % _NBUF
        if cps[p] is not None:
            cps[p].wait()
            base = wid * _ROWS_PER_W + (j - _NBUF) * _CHUNK
            pltpu.sync_copy(bufs[p], out_hbm.at[pl.ds(base, _CHUNK)])
        cps[p] = pltpu.async_copy(t_hbm.at[idx_v.at[j]], bufs[p], sems[p])
    for j in range(_NCHUNK - _NBUF, _NCHUNK):
        p = j % _NBUF
        cps[p].wait()
        base = wid * _ROWS_PER_W + j * _CHUNK
        pltpu.sync_copy(bufs[p], out_hbm.at[pl.ds(base, _CHUNK)])


@functools.cache
def _get_sc_gather():
    return pl.kernel(
        _sc_gather_body,
        out_type=jax.ShapeDtypeStruct((ROWS, CP), jnp.uint32),
        mesh=plsc.VectorSubcoreMesh(core_axis_name="c", subcore_axis_name="s"),
        scratch_types=[
            pltpu.VMEM((_NCHUNK, _CHUNK), jnp.int32),
            pltpu.VMEM((_CHUNK, CP), jnp.uint32),
            pltpu.VMEM((_CHUNK, CP), jnp.uint32),
            pltpu.VMEM((_CHUNK, CP), jnp.uint32),
            pltpu.SemaphoreType.DMA,
            pltpu.SemaphoreType.DMA,
            pltpu.SemaphoreType.DMA,
        ],
    )


_BH = 48
_NH = H // _BH


def _tc_transpose_body(f1_ref, f2_ref, out_ref):
    eye = jnp.eye(C, dtype=jnp.bfloat16)

    def slab_bits(f_ref):
        b = f_ref[0].reshape(C, _BH * W).astype(jnp.bfloat16)
        xt = lax.dot_general(b, eye, (((0,), (0,)), ((), ())),
                             preferred_element_type=jnp.float32)
        return lax.bitcast_convert_type(xt, jnp.uint32) >> 16

    w = slab_bits(f1_ref) | (slab_bits(f2_ref) << 16)
    out_ref[...] = jnp.concatenate(
        [w, jnp.zeros((_BH * W, CP - C), jnp.uint32)], axis=1)


def _tc_transpose(f1, f2):
    return pl.pallas_call(
        _tc_transpose_body,
        grid=(NB, _NH),
        in_specs=[
            pl.BlockSpec((1, C, _BH, W), lambda b, h: (b, 0, h, 0)),
            pl.BlockSpec((1, C, _BH, W), lambda b, h: (b, 0, h, 0)),
        ],
        out_specs=pl.BlockSpec((_BH * W, CP), lambda b, h: (b * _NH + h, 0)),
        out_shape=jax.ShapeDtypeStruct((NB * H * W, CP), jnp.uint32),
    )(f1, f2)


_ABLK = 512
_NBLK = NB * A // _ABLK


def _unpack(w):
    g1 = lax.bitcast_convert_type(w << 16, jnp.float32)
    g2 = lax.bitcast_convert_type(w & jnp.uint32(0xFFFF0000), jnp.float32)
    return g1, g2


def _tc_cosine_body(c_ref, n_ref, out_ref):
    ones = jnp.ones((CP, 128), jnp.bfloat16)

    def rowsums(p):
        return lax.dot_general(p.astype(jnp.bfloat16), ones,
                               (((1,), (0,)), ((), ())),
                               preferred_element_type=jnp.float32)

    def unit(g):
        return g * lax.rsqrt(rowsums(g * g))

    c1, c2 = _unpack(c_ref[...])
    n1, n2 = _unpack(n_ref[...])

    def sims(c, n):
        cb = jnp.broadcast_to(unit(c)[:, None, :], (_ABLK, K - 1, CP))
        pd = unit(n) * cb.reshape(_ABLK * (K - 1), CP)
        return rowsums(pd)

    part = jnp.sum(jnp.abs(sims(c1, n1) - sims(c2, n2))) / 128.0
    out_ref[pl.ds(pl.program_id(0), 1), :] = jnp.full((1, 128), part, jnp.float32)


def _tc_cosine(g):
    out = pl.pallas_call(
        _tc_cosine_body,
        grid=(_NBLK,),
        in_specs=[
            pl.BlockSpec((_ABLK, CP), lambda i: (i, 0)),
            pl.BlockSpec((_ABLK * (K - 1), CP), lambda i: (i + _NBLK // 8, 0)),
        ],
        out_specs=pl.BlockSpec((_NBLK, 128), lambda i: (0, 0)),
        out_shape=jax.ShapeDtypeStruct((_NBLK, 128), jnp.float32),
    )(g, g)
    return jnp.sum(out[:, 0])


def kernel(feat_list_1, feat_list_2, index_list):
    n = feat_list_1.shape[0]
    total = jnp.float32(0.0)
    for i in range(n):
        idx = index_list[i].astype(jnp.int32)
        q = idx[..., 0] * W + idx[..., 1]
        q = q + (jnp.arange(NB, dtype=jnp.int32) * (H * W)).reshape(NB, 1, 1)
        q = jnp.concatenate([q[..., 0].reshape(-1), q[..., 1:].reshape(-1)])
        q = q.reshape(_NW, _NCHUNK, _CHUNK)
        t = _tc_transpose(feat_list_1[i], feat_list_2[i])
        g = _get_sc_gather()(t, q)
        total = total + _tc_cosine(g) / (NB * A * 8)
    return total / n


# speedup vs baseline: 1.6791x; 1.0156x over previous
import functools

import jax
import jax.numpy as jnp
from jax import lax
from jax.experimental import pallas as pl
from jax.experimental.pallas import tpu as pltpu
from jax.experimental.pallas import tpu_sc as plsc

H = 384
W = 384
C = 96
A = 4096
NB = 2
K = 9
ROWS = NB * A * K
CP = 128

_NW = 32
_CHUNK = 128
_ROWS_PER_W = ROWS // _NW
_NCHUNK = _ROWS_PER_W // _CHUNK


_NBUF = 3


def _sc_gather_body(t_hbm, idx_hbm, out_hbm, idx_v, buf0, buf1, buf2,
                    sem0, sem1, sem2):
    wid = lax.axis_index("s") * 2 + lax.axis_index("c")
    pltpu.sync_copy(idx_hbm.at[wid], idx_v)
    bufs = (buf0, buf1, buf2)
    sems = (sem0, sem1, sem2)
    cps = [None] * _NBUF
    for j in range(_NCHUNK):
        p = j % _NBUF
        if cps[p] is not None:
            cps[p].wait()
            base = wid * _ROWS_PER_W + (j - _NBUF) * _CHUNK
            pltpu.sync_copy(bufs[p], out_hbm.at[pl.ds(base, _CHUNK)])
        cps[p] = pltpu.async_copy(t_hbm.at[idx_v.at[j]], bufs[p], sems[p])
    for j in range(_NCHUNK - _NBUF, _NCHUNK):
        p = j % _NBUF
        cps[p].wait()
        base = wid * _ROWS_PER_W + j * _CHUNK
        pltpu.sync_copy(bufs[p], out_hbm.at[pl.ds(base, _CHUNK)])


@functools.cache
def _get_sc_gather():
    return pl.kernel(
        _sc_gather_body,
        out_type=jax.ShapeDtypeStruct((ROWS, CP), jnp.uint32),
        mesh=plsc.VectorSubcoreMesh(core_axis_name="c", subcore_axis_name="s"),
        scratch_types=[
            pltpu.VMEM((_NCHUNK, _CHUNK), jnp.int32),
            pltpu.VMEM((_CHUNK, CP), jnp.uint32),
            pltpu.VMEM((_CHUNK, CP), jnp.uint32),
            pltpu.VMEM((_CHUNK, CP), jnp.uint32),
            pltpu.SemaphoreType.DMA,
            pltpu.SemaphoreType.DMA,
            pltpu.SemaphoreType.DMA,
        ],
    )


_BH = 48
_NH = H // _BH


def _tc_transpose_body(f1_ref, f2_ref, out_ref):
    eye = jnp.eye(C, dtype=jnp.bfloat16)

    def slab_bits(f_ref):
        b = f_ref[0].reshape(C, _BH * W).astype(jnp.bfloat16)
        xt = lax.dot_general(b, eye, (((0,), (0,)), ((), ())),
                             preferred_element_type=jnp.float32)
        return lax.bitcast_convert_type(xt, jnp.uint32) >> 16

    w = slab_bits(f1_ref) | (slab_bits(f2_ref) << 16)
    out_ref[...] = jnp.concatenate(
        [w, jnp.zeros((_BH * W, CP - C), jnp.uint32)], axis=1)


def _tc_transpose(f1, f2):
    return pl.pallas_call(
        _tc_transpose_body,
        grid=(NB, _NH),
        in_specs=[
            pl.BlockSpec((1, C, _BH, W), lambda b, h: (b, 0, h, 0)),
            pl.BlockSpec((1, C, _BH, W), lambda b, h: (b, 0, h, 0)),
        ],
        out_specs=pl.BlockSpec((_BH * W, CP), lambda b, h: (b * _NH + h, 0)),
        out_shape=jax.ShapeDtypeStruct((NB * H * W, CP), jnp.uint32),
    )(f1, f2)


_ABLK = 1024
_NBLK = NB * A // _ABLK


def _unpack(w):
    g1 = lax.bitcast_convert_type(w << 16, jnp.float32)
    g2 = lax.bitcast_convert_type(w & jnp.uint32(0xFFFF0000), jnp.float32)
    return g1, g2


def _tc_cosine_body(c_ref, n_ref, out_ref):
    ones = jnp.ones((CP, 128), jnp.bfloat16)

    def rowsums(p):
        return lax.dot_general(p.astype(jnp.bfloat16), ones,
                               (((1,), (0,)), ((), ())),
                               preferred_element_type=jnp.float32)

    def unit(g):
        return g * lax.rsqrt(rowsums(g * g))

    c1, c2 = _unpack(c_ref[...])
    n1, n2 = _unpack(n_ref[...])

    def sims(c, n):
        cb = jnp.broadcast_to(unit(c)[:, None, :], (_ABLK, K - 1, CP))
        pd = unit(n) * cb.reshape(_ABLK * (K - 1), CP)
        return rowsums(pd)

    part = jnp.sum(jnp.abs(sims(c1, n1) - sims(c2, n2))) / 128.0
    out_ref[pl.ds(pl.program_id(0), 1), :] = jnp.full((1, 128), part, jnp.float32)


def _tc_cosine(g):
    out = pl.pallas_call(
        _tc_cosine_body,
        grid=(_NBLK,),
        in_specs=[
            pl.BlockSpec((_ABLK, CP), lambda i: (i, 0)),
            pl.BlockSpec((_ABLK * (K - 1), CP), lambda i: (i + _NBLK // 8, 0)),
        ],
        out_specs=pl.BlockSpec((_NBLK, 128), lambda i: (0, 0)),
        out_shape=jax.ShapeDtypeStruct((_NBLK, 128), jnp.float32),
    )(g, g)
    return jnp.sum(out[:, 0])


def kernel(feat_list_1, feat_list_2, index_list):
    n = feat_list_1.shape[0]
    total = jnp.float32(0.0)
    for i in range(n):
        idx = index_list[i].astype(jnp.int32)
        q = idx[..., 0] * W + idx[..., 1]
        q = q + (jnp.arange(NB, dtype=jnp.int32) * (H * W)).reshape(NB, 1, 1)
        q = jnp.concatenate([q[..., 0].reshape(-1), q[..., 1:].reshape(-1)])
        q = q.reshape(_NW, _NCHUNK, _CHUNK)
        t = _tc_transpose(feat_list_1[i], feat_list_2[i])
        g = _get_sc_gather()(t, q)
        total = total + _tc_cosine(g) / (NB * A * 8)
    return total / n
